# baseline (speedup 1.0000x reference)
import jax
import jax.numpy as jnp
from jax import lax
from jax.experimental import pallas as pl
from jax.experimental.pallas import tpu as pltpu
from jax.experimental.pallas import tpu_sc as plsc

BATCH = 16384
DIM = 16
SLAB = 128
VOCAB8 = 125000
NC = 2
NS = 16
NW = NC * NS
B_PER_W = BATCH // NW
HALF = B_PER_W // 2
CHUNK = 128
GRP = 16
N_GRP = HALF // GRP

TCOLS = 16384
TGRID = -(-1000000 // TCOLS)


def _tr_body(tab_t_ref, out_ref, scr_ref):
    scr_ref[...] = tab_t_ref[...].T
    out_ref[...] = jnp.concatenate([scr_ref[s::8, :] for s in range(8)],
                                   axis=1)


def _transpose_tc(tab_t):
    return pl.pallas_call(
        _tr_body,
        grid=(TGRID,),
        in_specs=[pl.BlockSpec((DIM, TCOLS), lambda k: (0, k))],
        out_specs=pl.BlockSpec((TCOLS // 8, SLAB), lambda k: (k, 0)),
        out_shape=jax.ShapeDtypeStruct((VOCAB8, SLAB), jnp.float32),
        scratch_shapes=[pltpu.VMEM((TCOLS, DIM), jnp.float32)],
    )(tab_t)


def _lane_perm(x, idx):
    return lax.gather(
        x, idx[:, None],
        lax.GatherDimensionNumbers(
            offset_dims=(), collapsed_slice_dims=(0,), start_index_map=(0,)),
        (1,), mode=lax.GatherScatterMode.PROMISE_IN_BOUNDS)


def _body(u_hbm, v_hbm, tab_hbm, out_hbm, idx_u, idx_v, slab_u, slab_v,
          slabs_u, slabs_v, out_buf, sem):
    wid = lax.axis_index("s") * NC + lax.axis_index("c")
    base = wid * B_PER_W

    pltpu.sync_copy(u_hbm.at[pl.ds(base, B_PER_W)], idx_u)
    pltpu.sync_copy(v_hbm.at[pl.ds(base, B_PER_W)], idx_v)

    def shift_step(i, carry):
        off = i * GRP
        slab_u[pl.ds(off, GRP)] = lax.shift_right_logical(
            idx_u[pl.ds(off, GRP)], 3)
        slab_v[pl.ds(off, GRP)] = lax.shift_right_logical(
            idx_v[pl.ds(off, GRP)], 3)
        return carry

    lax.fori_loop(0, B_PER_W // GRP, shift_step, 0)

    lanes = lax.iota(jnp.int32, GRP)

    for h in range(2):
        hoff = h * HALF
        copies = []
        for c in range(HALF // CHUNK):
            sl = pl.ds(hoff + c * CHUNK, CHUNK)
            dsl = pl.ds(c * CHUNK, CHUNK)
            copies.append(pltpu.async_copy(
                tab_hbm.at[slab_u.at[sl]], slabs_u.at[dsl], sem))
            copies.append(pltpu.async_copy(
                tab_hbm.at[slab_v.at[sl]], slabs_v.at[dsl], sem))
        for cp in copies:
            cp.wait()

        def grp_step(g, carry):
            off = g * GRP
            su_vec = idx_u[pl.ds(hoff + off, GRP)] & 7
            sv_vec = idx_v[pl.ds(hoff + off, GRP)] & 7
            r = jnp.zeros((GRP,), jnp.float32)
            for j in range(GRP):
                su = su_vec[j]
                sv = sv_vec[j]
                ur = slabs_u[off + j, pl.ds(su * DIM, DIM)]
                vr = slabs_v[off + j, pl.ds(sv * DIM, DIM)]
                x = ur * vr
                for sh in (8, 4, 2, 1):
                    x = x + _lane_perm(x, lanes ^ sh)
                r = jnp.where(lanes == j, x, r)
            out_buf[pl.ds(hoff + off, GRP)] = 1.0 / (1.0 + jnp.exp(-r))
            return carry

        lax.fori_loop(0, N_GRP, grp_step, 0)

    pltpu.sync_copy(out_buf, out_hbm.at[pl.ds(base, B_PER_W)])


def _gather_sc(u, v, tab8):
    mesh = plsc.VectorSubcoreMesh(core_axis_name="c", subcore_axis_name="s")
    kfn = pl.kernel(
        _body,
        out_type=jax.ShapeDtypeStruct((BATCH,), jnp.float32),
        mesh=mesh,
        scratch_types=[
            pltpu.VMEM((B_PER_W,), jnp.int32),
            pltpu.VMEM((B_PER_W,), jnp.int32),
            pltpu.VMEM((B_PER_W,), jnp.int32),
            pltpu.VMEM((B_PER_W,), jnp.int32),
            pltpu.VMEM((HALF, SLAB), jnp.float32),
            pltpu.VMEM((HALF, SLAB), jnp.float32),
            pltpu.VMEM((B_PER_W,), jnp.float32),
            pltpu.SemaphoreType.DMA,
        ],
    )
    return kfn(u, v, tab8)


@jax.jit
def _run_full(u, v, tab_t):
    return _gather_sc(u, v, _transpose_tc(tab_t))


def kernel(u, v, embed_weight):
    return _run_full(u.astype(jnp.int32), v.astype(jnp.int32),
                     embed_weight.T)


# speedup vs baseline: 1.0067x; 1.0067x over previous
import jax
import jax.numpy as jnp
from jax import lax
from jax.experimental import pallas as pl
from jax.experimental.pallas import tpu as pltpu
from jax.experimental.pallas import tpu_sc as plsc

BATCH = 16384
DIM = 16
SLAB = 128
VOCAB8 = 125000
NC = 2
NS = 16
NW = NC * NS
B_PER_W = BATCH // NW
HALF = B_PER_W // 2
CHUNK = 128
GRP = 16
N_GRP = HALF // GRP

TCOLS = 8192
TGRID = -(-1000000 // TCOLS)


def _tr_body(tab_t_ref, out_ref, scr_ref):
    scr_ref[...] = tab_t_ref[...].T
    out_ref[...] = jnp.concatenate([scr_ref[s::8, :] for s in range(8)],
                                   axis=1)


def _transpose_tc(tab_t):
    return pl.pallas_call(
        _tr_body,
        grid=(TGRID,),
        in_specs=[pl.BlockSpec((DIM, TCOLS), lambda k: (0, k))],
        out_specs=pl.BlockSpec((TCOLS // 8, SLAB), lambda k: (k, 0)),
        out_shape=jax.ShapeDtypeStruct((VOCAB8, SLAB), jnp.float32),
        scratch_shapes=[pltpu.VMEM((TCOLS, DIM), jnp.float32)],
    )(tab_t)


def _lane_perm(x, idx):
    return lax.gather(
        x, idx[:, None],
        lax.GatherDimensionNumbers(
            offset_dims=(), collapsed_slice_dims=(0,), start_index_map=(0,)),
        (1,), mode=lax.GatherScatterMode.PROMISE_IN_BOUNDS)


def _body(u_hbm, v_hbm, tab_hbm, out_hbm, idx_u, idx_v, slab_u, slab_v,
          slabs_u, slabs_v, out_buf, sem):
    wid = lax.axis_index("s") * NC + lax.axis_index("c")
    base = wid * B_PER_W

    pltpu.sync_copy(u_hbm.at[pl.ds(base, B_PER_W)], idx_u)
    pltpu.sync_copy(v_hbm.at[pl.ds(base, B_PER_W)], idx_v)

    def shift_step(i, carry):
        off = i * GRP
        slab_u[pl.ds(off, GRP)] = lax.shift_right_logical(
            idx_u[pl.ds(off, GRP)], 3)
        slab_v[pl.ds(off, GRP)] = lax.shift_right_logical(
            idx_v[pl.ds(off, GRP)], 3)
        return carry

    lax.fori_loop(0, B_PER_W // GRP, shift_step, 0)

    lanes = lax.iota(jnp.int32, GRP)

    for h in range(2):
        hoff = h * HALF
        copies = []
        for c in range(HALF // CHUNK):
            sl = pl.ds(hoff + c * CHUNK, CHUNK)
            dsl = pl.ds(c * CHUNK, CHUNK)
            copies.append(pltpu.async_copy(
                tab_hbm.at[slab_u.at[sl]], slabs_u.at[dsl], sem))
            copies.append(pltpu.async_copy(
                tab_hbm.at[slab_v.at[sl]], slabs_v.at[dsl], sem))
        for cp in copies:
            cp.wait()

        def grp_step(g, carry):
            off = g * GRP
            su_vec = idx_u[pl.ds(hoff + off, GRP)] & 7
            sv_vec = idx_v[pl.ds(hoff + off, GRP)] & 7
            r = jnp.zeros((GRP,), jnp.float32)
            for j in range(GRP):
                su = su_vec[j]
                sv = sv_vec[j]
                ur = slabs_u[off + j, pl.ds(su * DIM, DIM)]
                vr = slabs_v[off + j, pl.ds(sv * DIM, DIM)]
                x = ur * vr
                for sh in (8, 4, 2, 1):
                    x = x + _lane_perm(x, lanes ^ sh)
                r = jnp.where(lanes == j, x, r)
            out_buf[pl.ds(hoff + off, GRP)] = 1.0 / (1.0 + jnp.exp(-r))
            return carry

        lax.fori_loop(0, N_GRP, grp_step, 0)

    pltpu.sync_copy(out_buf, out_hbm.at[pl.ds(base, B_PER_W)])


def _gather_sc(u, v, tab8):
    mesh = plsc.VectorSubcoreMesh(core_axis_name="c", subcore_axis_name="s")
    kfn = pl.kernel(
        _body,
        out_type=jax.ShapeDtypeStruct((BATCH,), jnp.float32),
        mesh=mesh,
        scratch_types=[
            pltpu.VMEM((B_PER_W,), jnp.int32),
            pltpu.VMEM((B_PER_W,), jnp.int32),
            pltpu.VMEM((B_PER_W,), jnp.int32),
            pltpu.VMEM((B_PER_W,), jnp.int32),
            pltpu.VMEM((HALF, SLAB), jnp.float32),
            pltpu.VMEM((HALF, SLAB), jnp.float32),
            pltpu.VMEM((B_PER_W,), jnp.float32),
            pltpu.SemaphoreType.DMA,
        ],
    )
    return kfn(u, v, tab8)


@jax.jit
def _run_full(u, v, tab_t):
    return _gather_sc(u, v, _transpose_tc(tab_t))


def kernel(u, v, embed_weight):
    return _run_full(u.astype(jnp.int32), v.astype(jnp.int32),
                     embed_weight.T)
